# baseline (speedup 1.0000x reference)
import functools

import jax
import jax.numpy as jnp
from jax import lax
from jax.experimental import pallas as pl
from jax.experimental.pallas import tpu as pltpu
from jax.experimental.pallas import tpu_sc as plsc

B = 2
L = 2048
D = 64
DG = 32
NH = 4
NB = 64
K = 64
BQ = 256
NEG = -1e9
THRESH = -1e8
NW = 32
RPW = B * L // NW
KV = L // 16
UNROLL = 8
LSHM = (0x3F, 0xFC0, 0x3F000, 0xFC0000)


def _hash_body(qt_ref, kt_ref, p0_ref, p1_ref, sig_ref, lsh_ref):
    for t, x_ref in enumerate((qt_ref, kt_ref)):
        xb = x_ref[0]
        for g, p_ref in enumerate((p0_ref, p1_ref)):
            xg = xb[g * DG:(g + 1) * DG, :]
            bits = (xg > 0).astype(jnp.int32)
            dio = lax.broadcasted_iota(jnp.int32, (DG, L), 0)
            sh = bits << (dio & 15)
            lo = jnp.sum(jnp.where(dio < 16, sh, 0), axis=0, keepdims=True)
            hi = jnp.sum(jnp.where(dio >= 16, sh, 0), axis=0, keepdims=True)
            sig = lo | (hi << 16)
            v = lax.dot_general(p_ref[...], xg, (((0,), (0,)), ((), ())),
                                preferred_element_type=jnp.float32)
            hb = (jnp.floor(v * 0.5).astype(jnp.int32)) & (NB - 1)
            hio = lax.broadcasted_iota(jnp.int32, (NH, L), 0)
            packed = jnp.sum(hb << (6 * hio), axis=0, keepdims=True)
            r = t * 2 + g
            sig_ref[0, r:r + 1, :] = sig
            lsh_ref[0, r:r + 1, :] = packed


def _hashes(qt, kt, p0, p1):
    return pl.pallas_call(
        _hash_body,
        grid=(B,),
        in_specs=[
            pl.BlockSpec((1, D, L), lambda b: (b, 0, 0)),
            pl.BlockSpec((1, D, L), lambda b: (b, 0, 0)),
            pl.BlockSpec((DG, NH), lambda b: (0, 0)),
            pl.BlockSpec((DG, NH), lambda b: (0, 0)),
        ],
        out_specs=[
            pl.BlockSpec((1, 4, L), lambda b: (b, 0, 0)),
            pl.BlockSpec((1, 4, L), lambda b: (b, 0, 0)),
        ],
        out_shape=[
            jax.ShapeDtypeStruct((B, 4, L), jnp.int32),
            jax.ShapeDtypeStruct((B, 4, L), jnp.int32),
        ],
    )(qt, kt, p0, p1)


def _sc_match(sig_hbm, lsh_hbm, cnt_hbm, mask_hbm,
              ksig_v, klsh_v, qsig_v, qlsh_v, cnt_v, row_v):
    wid = lax.axis_index("s") * 2 + lax.axis_index("c")
    b = wid // 16
    qbase = (wid % 16) * RPW
    pltpu.sync_copy(sig_hbm.at[b, pl.ds(2, 2)], ksig_v)
    pltpu.sync_copy(lsh_hbm.at[b, pl.ds(2, 2)], klsh_v)
    pltpu.sync_copy(sig_hbm.at[b, 0, pl.ds(qbase, RPW)],
                    qsig_v.at[pl.ds(0, RPW)])
    pltpu.sync_copy(sig_hbm.at[b, 1, pl.ds(qbase, RPW)],
                    qsig_v.at[pl.ds(RPW, RPW)])
    pltpu.sync_copy(lsh_hbm.at[b, 0, pl.ds(qbase, RPW)],
                    qlsh_v.at[pl.ds(0, RPW)])
    pltpu.sync_copy(lsh_hbm.at[b, 1, pl.ds(qbase, RPW)],
                    qlsh_v.at[pl.ds(RPW, RPW)])
    z16 = jnp.zeros((16,), jnp.int32)

    def row(qi, carry):
        qs0 = z16 + qsig_v[pl.ds(qi, 16)][0]
        qs1 = z16 + qsig_v[pl.ds(RPW + qi, 16)][0]

        def sweep(kj, acc):
            a = acc
            for u in range(UNROLL):
                sl = pl.ds((kj * UNROLL + u) * 16, 16)
                m = (ksig_v[0, sl] == qs0) | (ksig_v[1, sl] == qs1)
                a = a | jnp.where(m, 1, 0)
            return a
        hitw = lax.fori_loop(0, KV // UNROLL, sweep, z16)
        rc = jnp.sum(hitw)
        cnt_v[pl.ds(qi, 16)] = z16 + rc

        @pl.when(rc > 0)
        def _():
            ql0 = z16 + qlsh_v[pl.ds(qi, 16)][0]
            ql1 = z16 + qlsh_v[pl.ds(RPW + qi, 16)][0]

            def exact(kj, carry2):
                for u in range(UNROLL):
                    sl = pl.ds((kj * UNROLL + u) * 16, 16)
                    lx0 = klsh_v[0, sl] ^ ql0
                    lx1 = klsh_v[1, sl] ^ ql1
                    f0 = ((lx0 & LSHM[0]) == 0) | ((lx0 & LSHM[1]) == 0) | \
                         ((lx0 & LSHM[2]) == 0) | ((lx0 & LSHM[3]) == 0)
                    f1 = ((lx1 & LSHM[0]) == 0) | ((lx1 & LSHM[1]) == 0) | \
                         ((lx1 & LSHM[2]) == 0) | ((lx1 & LSHM[3]) == 0)
                    m = ((ksig_v[0, sl] == qs0) & f0) | \
                        ((ksig_v[1, sl] == qs1) & f1)
                    row_v[sl] = jnp.where(m, 1, 0)
                return carry2
            lax.fori_loop(0, KV // UNROLL, exact, 0)
            pltpu.sync_copy(row_v, mask_hbm.at[b, qbase + qi])
        return carry

    lax.fori_loop(0, RPW, row, 0)
    pltpu.sync_copy(cnt_v.at[pl.ds(0, RPW)], cnt_hbm.at[b, pl.ds(qbase, RPW)])


def _match(sig, lsh):
    mesh = plsc.VectorSubcoreMesh(core_axis_name="c", subcore_axis_name="s")
    f = functools.partial(
        pl.kernel, mesh=mesh,
        compiler_params=pltpu.CompilerParams(needs_layout_passes=False),
        out_type=[
            jax.ShapeDtypeStruct((B, L), jnp.int32),
            jax.ShapeDtypeStruct((B, L, L), jnp.int32),
        ],
        scratch_types=[
            pltpu.VMEM((2, L), jnp.int32),
            pltpu.VMEM((2, L), jnp.int32),
            pltpu.VMEM((2 * RPW + 128,), jnp.int32),
            pltpu.VMEM((2 * RPW + 128,), jnp.int32),
            pltpu.VMEM((RPW + 128,), jnp.int32),
            pltpu.VMEM((L,), jnp.int32),
        ],
    )(_sc_match)
    return f(sig, lsh)


def _score_body(q_ref, k_ref, cnt_ref, mask_ref, os_ref, oi_ref,
                msk_ref, done_ref):
    cntcol = cnt_ref[0]
    npos = jnp.sum(cntcol)
    os_ref[0] = jnp.full((BQ, K), NEG, jnp.float32)
    oi_ref[0] = jnp.full((BQ, K), -1, jnp.int32)
    done_ref[0] = jnp.where(npos > 0, 0, 1).astype(jnp.int32)

    @pl.when(npos > 0)
    def _():
        scores = lax.dot_general(q_ref[0], k_ref[0], (((1,), (1,)), ((), ())),
                                 preferred_element_type=jnp.float32)
        gate = (mask_ref[0] > 0) & (cntcol > 0)
        msk_ref[...] = jnp.where(gate, scores, NEG)

    kiota = lax.broadcasted_iota(jnp.int32, (BQ, L), 1)
    liota = lax.broadcasted_iota(jnp.int32, (BQ, K), 1)

    def step(j, carry):
        @pl.when(done_ref[0] == 0)
        def _():
            mm = msk_ref[...]
            m = jnp.max(mm, axis=1)
            bmax = jnp.max(m)

            @pl.when(bmax <= THRESH)
            def _():
                done_ref[0] = 1

            @pl.when(bmax > THRESH)
            def _():
                eq = mm == m[:, None]
                am = jnp.min(jnp.where(eq, kiota, L), axis=1)
                valid = m > THRESH
                sc = jnp.where(valid, m, NEG)
                ix = jnp.where(valid, am, -1)
                os_ref[0] = jnp.where(liota == j, sc[:, None], os_ref[0])
                oi_ref[0] = jnp.where(liota == j, ix[:, None], oi_ref[0])
                pop = (kiota == am[:, None]) & valid[:, None]
                msk_ref[...] = jnp.where(pop, NEG, mm)
        return carry

    lax.fori_loop(0, K, step, 0)


def _scores(q, k, cnt, mask):
    return pl.pallas_call(
        _score_body,
        grid=(B, L // BQ),
        in_specs=[
            pl.BlockSpec((1, BQ, D), lambda b, i: (b, i, 0)),
            pl.BlockSpec((1, L, D), lambda b, i: (b, 0, 0)),
            pl.BlockSpec((1, BQ, 1), lambda b, i: (b, i, 0)),
            pl.BlockSpec((1, BQ, L), lambda b, i: (b, i, 0)),
        ],
        out_specs=[
            pl.BlockSpec((1, BQ, K), lambda b, i: (b, i, 0)),
            pl.BlockSpec((1, BQ, K), lambda b, i: (b, i, 0)),
        ],
        out_shape=[
            jax.ShapeDtypeStruct((B, L, K), jnp.float32),
            jax.ShapeDtypeStruct((B, L, K), jnp.int32),
        ],
        scratch_shapes=[
            pltpu.VMEM((BQ, L), jnp.float32),
            pltpu.SMEM((1,), jnp.int32),
        ],
    )(q, k, cnt, mask)


@jax.jit
def _run(q, k, p0, p1):
    qt = jnp.transpose(q, (0, 2, 1))
    kt = jnp.transpose(k, (0, 2, 1))
    sig, lsh = _hashes(qt, kt, p0, p1)
    cnt, mask = _match(sig, lsh)
    return _scores(q, k, cnt[..., None], mask)


def kernel(query_up, key_up, lsh_proj_g0, lsh_proj_g1, head_idx=0):
    del head_idx
    return _run(query_up, key_up, lsh_proj_g0, lsh_proj_g1)
